# baseline (device time: 95688 ns/iter reference)
import jax
import jax.numpy as jnp
from jax import lax
from jax.experimental import pallas as pl
from jax.experimental.pallas import tpu as pltpu

N_DEV = 16
M_PER = 256
COMM_DTYPE = jnp.bfloat16
SUBS_PER_DIR = 4


def kernel(x, w_mat, scale_x, scale_w):
    m, k_per = x.shape
    _, n = w_mat.shape
    n_half = n // 2
    n_sub = n_half // SUBS_PER_DIR
    n_subs = 2 * SUBS_PER_DIR

    def body(x_ref, w_ref, sx_ref, sw_ref, out_ref, *scr):
        x_bf, w_bf = x_ref, w_ref
        comms = scr[:n_subs]
        sems = scr[n_subs:]

        my = lax.axis_index("i")
        m4 = my % 4
        r = jnp.where(
            m4 == 0, my // 4,
            jnp.where(m4 == 3, (15 - my) // 4 + 4,
                      jnp.where(m4 == 2, (my - 2) // 4 + 8,
                                (13 - my) // 4 + 12)))

        def ringfn(p):
            p = p % N_DEV
            return jnp.where(
                p < 4, 4 * p,
                jnp.where(p < 8, 31 - 4 * p,
                          jnp.where(p < 12, 4 * p - 30, 61 - 4 * p)))

        right = ringfn(r + 1)
        left = ringfn(r - 1)
        cids_a = [ringfn(r - 1 - t) for t in range(N_DEV)]
        cids_b = [ringfn(r + 1 + t) for t in range(N_DEV)]

        subs = []
        for j in range(SUBS_PER_DIR):
            subs.append((comms[2 * j], sems[4 * j], sems[4 * j + 1],
                         right, j * n_sub, cids_a))
            subs.append((comms[2 * j + 1], sems[4 * j + 2], sems[4 * j + 3],
                         left, n_half + j * n_sub, cids_b))

        def partial(cid, col0):
            xs = x_bf[pl.ds(cid * M_PER, M_PER), :]
            ws = w_bf[:, col0:col0 + n_sub]
            return lax.dot_general(
                xs, ws, (((1,), (0,)), ((), ())),
                preferred_element_type=jnp.float32,
            )

        def make_rdma(comm, ssem, rsem, dev, t):
            return pltpu.make_async_remote_copy(
                src_ref=comm.at[t % 2],
                dst_ref=comm.at[(t + 1) % 2],
                send_sem=ssem.at[t % 2],
                recv_sem=rsem.at[(t + 1) % 2],
                device_id=(dev,),
                device_id_type=pl.DeviceIdType.MESH,
            )

        for comm, ssem, rsem, dev, col0, cids in subs:
            comm[0, :, :] = partial(cids[0], col0).astype(COMM_DTYPE)

        barrier_sem = pltpu.get_barrier_semaphore()
        for nbr in (left, right):
            pl.semaphore_signal(
                barrier_sem, inc=1,
                device_id=(nbr,), device_id_type=pl.DeviceIdType.MESH,
            )
        pl.semaphore_wait(barrier_sem, 2)

        inflight = []
        for comm, ssem, rsem, dev, col0, cids in subs:
            rdma = make_rdma(comm, ssem, rsem, dev, 0)
            rdma.start()
            inflight.append(rdma)

        for t in range(1, N_DEV - 1):
            for i, (comm, ssem, rsem, dev, col0, cids) in enumerate(subs):
                part = partial(cids[t], col0).astype(COMM_DTYPE)
                inflight[i].wait()
                slot = t % 2
                comm[slot, :, :] = comm[slot, :, :] + part
                rdma = make_rdma(comm, ssem, rsem, dev, t)
                rdma.start()
                inflight[i] = rdma

        scale = sx_ref[0] * sw_ref[0]
        for i, (comm, ssem, rsem, dev, col0, cids) in enumerate(subs):
            part = partial(cids[N_DEV - 1], col0)
            inflight[i].wait()
            acc = comm[(N_DEV - 1) % 2, :, :].astype(jnp.float32) + part
            y = acc * scale
            out_ref[:, col0:col0 + n_sub] = y * jax.nn.sigmoid(
                jnp.clip(y, -60.0, 60.0))

    scratch = [pltpu.VMEM((2, M_PER, n_sub), COMM_DTYPE)
               for _ in range(n_subs)]
    scratch += [pltpu.SemaphoreType.DMA((2,)) for _ in range(2 * n_subs)]

    return pl.pallas_call(
        body,
        out_shape=jax.ShapeDtypeStruct((M_PER, n), jnp.float32),
        in_specs=[
            pl.BlockSpec(memory_space=pltpu.VMEM),
            pl.BlockSpec(memory_space=pltpu.VMEM),
            pl.BlockSpec(memory_space=pltpu.SMEM),
            pl.BlockSpec(memory_space=pltpu.SMEM),
        ],
        out_specs=pl.BlockSpec(memory_space=pltpu.VMEM),
        scratch_shapes=scratch,
        compiler_params=pltpu.CompilerParams(collective_id=0),
    )(x.astype(jnp.bfloat16), w_mat.astype(jnp.bfloat16), scale_x, scale_w)


# device time: 95378 ns/iter; 1.0033x vs baseline; 1.0033x over previous
import jax
import jax.numpy as jnp
from jax import lax
from jax.experimental import pallas as pl
from jax.experimental.pallas import tpu as pltpu

N_DEV = 16
M_PER = 256
COMM_DTYPE = jnp.bfloat16
SUBS_PER_DIR = 4


def kernel(x, w_mat, scale_x, scale_w):
    m, k_per = x.shape
    _, n = w_mat.shape
    n_half = n // 2
    n_sub = n_half // SUBS_PER_DIR
    n_subs = 2 * SUBS_PER_DIR

    def body(x_ref, w_ref, sx_ref, sw_ref, out_ref, *scr):
        x_bf, w_bf = scr[0], scr[1]
        comms = scr[2:2 + n_subs]
        sems = scr[2 + n_subs:]

        my = lax.axis_index("i")
        m4 = my % 4
        r = jnp.where(
            m4 == 0, my // 4,
            jnp.where(m4 == 3, (15 - my) // 4 + 4,
                      jnp.where(m4 == 2, (my - 2) // 4 + 8,
                                (13 - my) // 4 + 12)))

        def ringfn(p):
            p = p % N_DEV
            return jnp.where(
                p < 4, 4 * p,
                jnp.where(p < 8, 31 - 4 * p,
                          jnp.where(p < 12, 4 * p - 30, 61 - 4 * p)))

        right = ringfn(r + 1)
        left = ringfn(r - 1)
        cids_a = [ringfn(r - 1 - t) for t in range(N_DEV)]
        cids_b = [ringfn(r + 1 + t) for t in range(N_DEV)]

        x_bf[:, :] = x_ref[:, :].astype(jnp.bfloat16)
        w_bf[:, :] = w_ref[:, :].astype(jnp.bfloat16)

        subs = []
        for j in range(SUBS_PER_DIR):
            subs.append((comms[2 * j], sems[4 * j], sems[4 * j + 1],
                         right, j * n_sub, cids_a))
            subs.append((comms[2 * j + 1], sems[4 * j + 2], sems[4 * j + 3],
                         left, n_half + j * n_sub, cids_b))

        def partial(cid, col0):
            xs = x_bf[pl.ds(cid * M_PER, M_PER), :]
            ws = w_bf[:, col0:col0 + n_sub]
            return lax.dot_general(
                xs, ws, (((1,), (0,)), ((), ())),
                preferred_element_type=jnp.float32,
            )

        def make_rdma(comm, ssem, rsem, dev, t):
            return pltpu.make_async_remote_copy(
                src_ref=comm.at[t % 2],
                dst_ref=comm.at[(t + 1) % 2],
                send_sem=ssem.at[t % 2],
                recv_sem=rsem.at[(t + 1) % 2],
                device_id=(dev,),
                device_id_type=pl.DeviceIdType.MESH,
            )

        for comm, ssem, rsem, dev, col0, cids in subs:
            comm[0, :, :] = partial(cids[0], col0).astype(COMM_DTYPE)

        barrier_sem = pltpu.get_barrier_semaphore()
        for nbr in (left, right):
            pl.semaphore_signal(
                barrier_sem, inc=1,
                device_id=(nbr,), device_id_type=pl.DeviceIdType.MESH,
            )
        pl.semaphore_wait(barrier_sem, 2)

        inflight = []
        for comm, ssem, rsem, dev, col0, cids in subs:
            rdma = make_rdma(comm, ssem, rsem, dev, 0)
            rdma.start()
            inflight.append(rdma)

        for t in range(1, N_DEV - 1):
            for i, (comm, ssem, rsem, dev, col0, cids) in enumerate(subs):
                part = partial(cids[t], col0).astype(COMM_DTYPE)
                inflight[i].wait()
                slot = t % 2
                comm[slot, :, :] = comm[slot, :, :] + part
                rdma = make_rdma(comm, ssem, rsem, dev, t)
                rdma.start()
                inflight[i] = rdma

        scale = sx_ref[0] * sw_ref[0]
        for i, (comm, ssem, rsem, dev, col0, cids) in enumerate(subs):
            part = partial(cids[N_DEV - 1], col0)
            inflight[i].wait()
            acc = comm[(N_DEV - 1) % 2, :, :].astype(jnp.float32) + part
            y = acc * scale
            out_ref[:, col0:col0 + n_sub] = y * jax.nn.sigmoid(
                jnp.clip(y, -60.0, 60.0))

    scratch = [
        pltpu.VMEM((m, k_per), jnp.bfloat16),
        pltpu.VMEM((k_per, n), jnp.bfloat16),
    ]
    scratch += [pltpu.VMEM((2, M_PER, n_sub), COMM_DTYPE)
                for _ in range(n_subs)]
    scratch += [pltpu.SemaphoreType.DMA((2,)) for _ in range(2 * n_subs)]

    return pl.pallas_call(
        body,
        out_shape=jax.ShapeDtypeStruct((M_PER, n), jnp.float32),
        in_specs=[
            pl.BlockSpec(memory_space=pltpu.VMEM),
            pl.BlockSpec(memory_space=pltpu.VMEM),
            pl.BlockSpec(memory_space=pltpu.SMEM),
            pl.BlockSpec(memory_space=pltpu.SMEM),
        ],
        out_specs=pl.BlockSpec(memory_space=pltpu.VMEM),
        scratch_shapes=scratch,
        compiler_params=pltpu.CompilerParams(collective_id=0),
    )(x, w_mat, scale_x, scale_w)
